# Initial kernel scaffold; baseline (speedup 1.0000x reference)
import jax
import jax.numpy as jnp
from jax.experimental import pallas as pl


def kernel(x, token_table, pos_table):
    raise NotImplementedError("write your pallas kernel here")


# speedup vs baseline: 3.1090x; 3.1090x over previous
import functools

import jax
import jax.numpy as jnp
from jax import lax
from jax.experimental import pallas as pl
from jax.experimental.pallas import tpu as pltpu
import jax.experimental.pallas.tpu_sc as plsc

MAXLEN = 200
EMBED = 64
NUM_CORES = 2
NUM_SUBCORES = 16
NUM_WORKERS = NUM_CORES * NUM_SUBCORES
LANES = 16


def _body(x_ref, tok_ref, pos_ref, out_ref, idx_v, rows_v, pos_v, sem):
  wid = lax.axis_index("s") * NUM_CORES + lax.axis_index("c")
  chunks_per_worker = x_ref.shape[0] // (MAXLEN * NUM_WORKERS)
  base_chunk = wid * chunks_per_worker

  pltpu.sync_copy(pos_ref, pos_v)

  def chunk(c, carry):
    row0 = (base_chunk + c) * MAXLEN
    pltpu.sync_copy(x_ref.at[pl.ds(row0, MAXLEN)], idx_v)
    ca = pltpu.async_copy(
        tok_ref.at[idx_v.at[pl.ds(0, 128)]], rows_v.at[pl.ds(0, 128)], sem)
    cb = pltpu.async_copy(
        tok_ref.at[idx_v.at[pl.ds(128, MAXLEN - 128)]],
        rows_v.at[pl.ds(128, MAXLEN - 128)], sem)
    ca.wait()
    cb.wait()

    def add_pos(t, carry2):
      for j in range(EMBED // LANES):
        v = pos_v[t, pl.ds(j * LANES, LANES)]
        plsc.addupdate(rows_v.at[t, pl.ds(j * LANES, LANES)], v)
      return carry2

    lax.fori_loop(0, MAXLEN, add_pos, None)
    pltpu.sync_copy(rows_v, out_ref.at[pl.ds(row0, MAXLEN)])
    return carry

  lax.fori_loop(0, chunks_per_worker, chunk, None)


def kernel(x, token_table, pos_table):
  batch, maxlen = x.shape
  rows = batch * maxlen
  x_flat = x.reshape(rows).astype(jnp.int32)
  mesh = plsc.VectorSubcoreMesh(core_axis_name="c", subcore_axis_name="s")
  out = pl.kernel(
      _body,
      out_type=jax.ShapeDtypeStruct((rows, EMBED), jnp.float32),
      mesh=mesh,
      compiler_params=pltpu.CompilerParams(use_tc_tiling_on_sc=False),
      scratch_types=[
          pltpu.VMEM((MAXLEN,), jnp.int32),
          pltpu.VMEM((MAXLEN, EMBED), jnp.float32),
          pltpu.VMEM((MAXLEN, EMBED), jnp.float32),
          pltpu.SemaphoreType.DMA,
      ],
  )(x_flat, token_table, pos_table)
  return out.reshape(batch, maxlen, EMBED)


# speedup vs baseline: 4.1375x; 1.3308x over previous
import jax
import jax.numpy as jnp
from jax import lax
from jax.experimental import pallas as pl
from jax.experimental.pallas import tpu as pltpu
import jax.experimental.pallas.tpu_sc as plsc

MAXLEN = 200
EMBED = 64
NUM_CORES = 2
NUM_SUBCORES = 16
NUM_WORKERS = NUM_CORES * NUM_SUBCORES
LANES = 16
NSLOT = 4
G0 = 128
G1 = MAXLEN - G0


def _body(x_ref, tok_ref, pos_ref, out_ref, idx_v, rows_v, pos_v,
          gsem, osem, isem):
  wid = lax.axis_index("s") * NUM_CORES + lax.axis_index("c")
  n_chunks = x_ref.shape[0] // (MAXLEN * NUM_WORKERS)
  base_chunk = wid * n_chunks

  def idx_start(c, s):
    row0 = (base_chunk + c) * MAXLEN
    return pltpu.async_copy(
        x_ref.at[pl.ds(row0, MAXLEN)], idx_v.at[s], isem.at[s])

  def gather_start(s):
    pltpu.async_copy(
        tok_ref.at[idx_v.at[s, pl.ds(0, G0)]],
        rows_v.at[s, pl.ds(0, G0)], gsem.at[s])
    pltpu.async_copy(
        tok_ref.at[idx_v.at[s, pl.ds(G0, G1)]],
        rows_v.at[s, pl.ds(G0, G1)], gsem.at[s])

  def gather_wait(s):
    pltpu.make_async_copy(
        tok_ref.at[idx_v.at[s, pl.ds(0, G0)]],
        rows_v.at[s, pl.ds(0, G0)], gsem.at[s]).wait()
    pltpu.make_async_copy(
        tok_ref.at[idx_v.at[s, pl.ds(G0, G1)]],
        rows_v.at[s, pl.ds(G0, G1)], gsem.at[s]).wait()

  def out_start(c, s):
    row0 = (base_chunk + c) * MAXLEN
    return pltpu.async_copy(
        rows_v.at[s], out_ref.at[pl.ds(row0, MAXLEN)], osem.at[s])

  def out_wait(c, s):
    row0 = (base_chunk + c) * MAXLEN
    pltpu.make_async_copy(
        rows_v.at[s], out_ref.at[pl.ds(row0, MAXLEN)], osem.at[s]).wait()

  pltpu.sync_copy(pos_ref, pos_v)

  for s in range(NSLOT):
    idx_start(s, s)
  for s in range(2):
    pltpu.make_async_copy(
        x_ref.at[pl.ds(0, MAXLEN)], idx_v.at[s], isem.at[s]).wait()
    gather_start(s)

  def outer(c4, carry):
    for s in range(NSLOT):
      c = c4 * NSLOT + s
      s2 = (s + 2) % NSLOT

      gather_wait(s)

      rows_s = rows_v.at[s]

      def add_body(t8, carry2):
        for r in range(8):
          t = t8 * 8 + r
          for j in range(EMBED // LANES):
            v = pos_v[t, pl.ds(j * LANES, LANES)]
            plsc.addupdate(rows_s.at[t, pl.ds(j * LANES, LANES)], v)
        return carry2

      lax.fori_loop(0, MAXLEN // 8, add_body, None)

      out_start(c, s)

      @pl.when(c + 2 < n_chunks)
      def _():
        pltpu.make_async_copy(
            x_ref.at[pl.ds(0, MAXLEN)], idx_v.at[s2], isem.at[s2]).wait()

        @pl.when(c >= 2)
        def _():
          out_wait(c - 2, s2)

        gather_start(s2)

      @pl.when(c + 4 < n_chunks)
      def _():
        idx_start(c + 4, s)
    return carry

  lax.fori_loop(0, n_chunks // NSLOT, outer, None)

  for s in range(NSLOT):
    out_wait(n_chunks - NSLOT + s, s)


def kernel(x, token_table, pos_table):
  batch, maxlen = x.shape
  rows = batch * maxlen
  x_flat = x.reshape(rows).astype(jnp.int32)
  mesh = plsc.VectorSubcoreMesh(core_axis_name="c", subcore_axis_name="s")
  out = pl.kernel(
      _body,
      out_type=jax.ShapeDtypeStruct((rows, EMBED), jnp.float32),
      mesh=mesh,
      compiler_params=pltpu.CompilerParams(use_tc_tiling_on_sc=False),
      scratch_types=[
          pltpu.VMEM((NSLOT, MAXLEN), jnp.int32),
          pltpu.VMEM((NSLOT, MAXLEN, EMBED), jnp.float32),
          pltpu.VMEM((MAXLEN, EMBED), jnp.float32),
          pltpu.SemaphoreType.DMA((NSLOT,)),
          pltpu.SemaphoreType.DMA((NSLOT,)),
          pltpu.SemaphoreType.DMA((NSLOT,)),
      ],
  )(x_flat, token_table, pos_table)
  return out.reshape(batch, maxlen, EMBED)
